# baseline (speedup 1.0000x reference)
import jax
import jax.numpy as jnp
from jax.experimental import pallas as pl
from jax.experimental.pallas import tpu as pltpu

N = 4096
D = 256
BM = 256
NB = N // BM


def _fused_gcn_kernel(adj_ref, x_ref, w1_ref, b1_ref, w2_ref, b2_ref,
                      o_ref, adjbf_ref, s1_ref, s2_ref):
    m = pl.program_id(0)

    @pl.when(m == 0)
    def _():
        s1_ref[...] = jnp.dot(
            x_ref[...], w1_ref[...], preferred_element_type=jnp.float32
        ).astype(jnp.bfloat16)

    ab = adj_ref[...].astype(jnp.bfloat16)
    adjbf_ref[pl.ds(m * BM, BM), :] = ab
    t = jnp.dot(ab, s1_ref[...], preferred_element_type=jnp.float32)
    h_m = jnp.maximum(t + b1_ref[...], 0.0).astype(jnp.bfloat16)
    s2_ref[pl.ds(m * BM, BM), :] = jnp.dot(
        h_m, w2_ref[...], preferred_element_type=jnp.float32
    ).astype(jnp.bfloat16)

    o_ref[pl.ds(m * BM, BM), :] = jnp.broadcast_to(b2_ref[...], (BM, D))
    for k in range(NB - 1):
        @pl.when(k < m)
        def _(k=k):
            o_ref[pl.ds(m * BM, BM), :] += jnp.dot(
                adjbf_ref[pl.ds(m * BM, BM), pl.ds(k * BM, BM)],
                s2_ref[pl.ds(k * BM, BM), :],
                preferred_element_type=jnp.float32,
            )

    for i in range(NB):
        @pl.when(i <= m)
        def _(i=i):
            o_ref[pl.ds(i * BM, BM), :] += jnp.dot(
                adjbf_ref[pl.ds(i * BM, BM), pl.ds(m * BM, BM)],
                s2_ref[pl.ds(m * BM, BM), :],
                preferred_element_type=jnp.float32,
            )


def kernel(x, adj, W1, b1, W2, b2):
    xb = x.astype(jnp.bfloat16)
    w1b = W1.astype(jnp.bfloat16)
    w2b = W2.astype(jnp.bfloat16)
    b1r = b1.reshape(1, D)
    b2r = b2.reshape(1, D)
    return pl.pallas_call(
        _fused_gcn_kernel,
        grid=(NB,),
        in_specs=[
            pl.BlockSpec((BM, N), lambda i: (i, 0)),
            pl.BlockSpec((N, D), lambda i: (0, 0)),
            pl.BlockSpec((D, D), lambda i: (0, 0)),
            pl.BlockSpec((1, D), lambda i: (0, 0)),
            pl.BlockSpec((D, D), lambda i: (0, 0)),
            pl.BlockSpec((1, D), lambda i: (0, 0)),
        ],
        out_specs=pl.BlockSpec((N, D), lambda i: (0, 0)),
        out_shape=jax.ShapeDtypeStruct((N, D), jnp.float32),
        scratch_shapes=[
            pltpu.VMEM((N, N), jnp.bfloat16),
            pltpu.VMEM((N, D), jnp.bfloat16),
            pltpu.VMEM((N, D), jnp.bfloat16),
        ],
    )(adj, xb, w1b, b1r, w2b, b2r)


# speedup vs baseline: 1.7217x; 1.7217x over previous
import jax
import jax.numpy as jnp
from jax.experimental import pallas as pl
from jax.experimental.pallas import tpu as pltpu

N = 4096
D = 256
BM = 512
NB = N // BM


def _fused_gcn_kernel(adj_ref, x_ref, w1_ref, b1_ref, w2_ref, b2_ref,
                      o_ref, adjbf_ref, s1_ref, s2_ref):
    m = pl.program_id(0)

    @pl.when(m == 0)
    def _():
        s1_ref[...] = jnp.dot(
            x_ref[...], w1_ref[...], preferred_element_type=jnp.float32
        ).astype(jnp.bfloat16)

    for c in range(NB):
        @pl.when(m == c)
        def _(c=c):
            r0, r1 = c * BM, (c + 1) * BM
            ab = adj_ref[...].astype(jnp.bfloat16)
            if c < NB - 1:
                adjbf_ref[r0:r1, :] = ab

            t = jnp.dot(ab, s1_ref[...], preferred_element_type=jnp.float32)
            h_c = jnp.maximum(t + b1_ref[...], 0.0).astype(jnp.bfloat16)
            s2_c = jnp.dot(
                h_c, w2_ref[...], preferred_element_type=jnp.float32
            ).astype(jnp.bfloat16)
            s2_ref[r0:r1, :] = s2_c

            init = jnp.broadcast_to(b2_ref[...], (BM, D)).astype(jnp.float32)
            if c > 0:
                init = init + jnp.dot(
                    ab[:, :r0], s2_ref[:r0, :],
                    preferred_element_type=jnp.float32,
                )
            o_ref[r0:r1, :] = init + jnp.dot(
                ab[:, r0:r1], s2_c, preferred_element_type=jnp.float32
            )
            if c > 0:
                o_ref[:r0, :] += jnp.dot(
                    adjbf_ref[:r0, r0:r1], s2_c,
                    preferred_element_type=jnp.float32,
                )


def kernel(x, adj, W1, b1, W2, b2):
    xb = x.astype(jnp.bfloat16)
    w1b = W1.astype(jnp.bfloat16)
    w2b = W2.astype(jnp.bfloat16)
    b1r = b1.reshape(1, D)
    b2r = b2.reshape(1, D)
    return pl.pallas_call(
        _fused_gcn_kernel,
        grid=(NB,),
        in_specs=[
            pl.BlockSpec((BM, N), lambda i: (i, 0)),
            pl.BlockSpec((N, D), lambda i: (0, 0)),
            pl.BlockSpec((D, D), lambda i: (0, 0)),
            pl.BlockSpec((1, D), lambda i: (0, 0)),
            pl.BlockSpec((D, D), lambda i: (0, 0)),
            pl.BlockSpec((1, D), lambda i: (0, 0)),
        ],
        out_specs=pl.BlockSpec((N, D), lambda i: (0, 0)),
        out_shape=jax.ShapeDtypeStruct((N, D), jnp.float32),
        scratch_shapes=[
            pltpu.VMEM((N - BM, N), jnp.bfloat16),
            pltpu.VMEM((N, D), jnp.bfloat16),
            pltpu.VMEM((N, D), jnp.bfloat16),
        ],
    )(adj, xb, w1b, b1r, w2b, b2r)
